# Initial kernel scaffold; baseline (speedup 1.0000x reference)
import jax
import jax.numpy as jnp
from jax.experimental import pallas as pl


def kernel(x, A, W1a, b1a, g1a, be1a, W2a, b2a, W1b, b1b, g1b, be1b, W2b, b2b):
    raise NotImplementedError("write your pallas kernel here")


# speedup vs baseline: 1.1754x; 1.1754x over previous
import functools

import jax
import jax.numpy as jnp
import numpy as np
from jax.experimental import pallas as pl
from jax.experimental.pallas import tpu as pltpu

N = 4096
NFEAT = 256
NHID = 256
OUT_DIM = 128
BN_EPS = 1e-5

I_BLK = 512


def _conv_body(a_ref, xb_ref, xres_ref, w1_ref, b1_ref, w2_ref, b2_ref,
               *out_refs, relu_out, dual_out):
    mask = (a_ref[...] > 0.0).astype(jnp.bfloat16)
    aggr = jax.lax.dot_general(
        mask, xb_ref[...], (((0,), (0,)), ((), ())),
        preferred_element_type=jnp.float32)
    aggr = aggr + xres_ref[...]
    h = jnp.dot(aggr.astype(jnp.bfloat16), w1_ref[...],
                preferred_element_type=jnp.float32) + b1_ref[...]
    h = jnp.maximum(h, 0.0)
    o = jnp.dot(h.astype(jnp.bfloat16), w2_ref[...],
                preferred_element_type=jnp.float32) + b2_ref[...]
    if relu_out:
        o = jnp.maximum(o, 0.0)
    out_refs[0][...] = o
    if dual_out:
        out_refs[1][...] = o.astype(jnp.bfloat16)


def _gin_conv(A, xb, xres, w1, b1, w2, b2, out_dim, relu_out, dual_out):
    n_i = N // I_BLK
    full = lambda shape: pl.BlockSpec(shape, lambda i: (0, 0))
    in_specs = [
        pl.BlockSpec((N, I_BLK), lambda i: (0, i)),
        full((N, NFEAT)),
        pl.BlockSpec((I_BLK, NFEAT), lambda i: (i, 0)),
        full(w1.shape),
        full(b1.shape),
        full(w2.shape),
        full(b2.shape),
    ]
    out_shape = [jax.ShapeDtypeStruct((N, out_dim), jnp.float32)]
    out_specs = [pl.BlockSpec((I_BLK, out_dim), lambda i: (i, 0))]
    if dual_out:
        out_shape.append(jax.ShapeDtypeStruct((N, out_dim), jnp.bfloat16))
        out_specs.append(pl.BlockSpec((I_BLK, out_dim), lambda i: (i, 0)))
    return pl.pallas_call(
        functools.partial(_conv_body, relu_out=relu_out, dual_out=dual_out),
        grid=(n_i,),
        in_specs=in_specs,
        out_specs=out_specs,
        out_shape=out_shape,
    )(A, xb, xres, w1, b1, w2, b2)


def kernel(x, A, W1a, b1a, g1a, be1a, W2a, b2a, W1b, b1b, g1b, be1b, W2b, b2b):
    inv = np.float32(1.0 / np.sqrt(1.0 + BN_EPS))
    gs_a = g1a * inv
    w1a = (W1a * gs_a[None, :]).astype(jnp.bfloat16)
    c1a = (b1a * gs_a + be1a)[None, :]
    gs_b = g1b * inv
    w1b = (W1b * gs_b[None, :]).astype(jnp.bfloat16)
    c1b = (b1b * gs_b + be1b)[None, :]
    w2a = W2a.astype(jnp.bfloat16)
    w2b = W2b.astype(jnp.bfloat16)

    xb = x.astype(jnp.bfloat16)
    H, Hb = _gin_conv(A, xb, x, w1a, c1a, w2a, b2a[None, :],
                      out_dim=NHID, relu_out=True, dual_out=True)
    out, = _gin_conv(A, Hb, H, w1b, c1b, w2b, b2b[None, :],
                     out_dim=OUT_DIM, relu_out=False, dual_out=False)
    return out
